# baseline (speedup 1.0000x reference)
import jax
import jax.numpy as jnp
from jax.experimental import pallas as pl
from jax.experimental.pallas import tpu as pltpu

_LIMIT = 8
_THRESHOLD = 0.9
_EPS = 1e-5


def _shift(a, c):
    if c == 0:
        return a
    return pltpu.roll(a, a.shape[1] - c, axis=1)


def _fused_kernel(x_ref, rm_ref, worst_ref, incl_ref, band_ref):
    i = pl.program_id(0)
    b = pl.num_programs(0)
    s_len = x_ref.shape[1]

    x = x_ref[0]
    cols = []
    for d in range(_LIMIT):
        sh = x if d == 0 else pltpu.roll(x, s_len - d, axis=0)
        c = jnp.sum(x * sh, axis=1, keepdims=True)
        if d > 0:
            sub = jax.lax.broadcasted_iota(jnp.int32, (s_len, 1), 0)
            c = jnp.where(sub < s_len - d, c, 0.0)
        cols.append(c)
    band_ref[i] = jnp.concatenate(cols, axis=1)

    @pl.when(i == b - 1)
    def _windows():
        _window_stage(band_ref, rm_ref, worst_ref, incl_ref)


def _window_stage(band_ref, rm_ref, worst_ref, incl_ref):
    b, s_len, _ = band_ref.shape
    m = jnp.concatenate(
        [band_ref[bi][:, d:d + 1] for d in range(_LIMIT) for bi in range(b)],
        axis=1)
    mt = m.T
    a = [jax.lax.slice_in_dim(mt, d * b, (d + 1) * b, axis=0)
         for d in range(_LIMIT)]

    rmf = rm_ref[...]
    n = a[0]
    gns = [_shift(jnp.maximum(jnp.sqrt(n), _EPS), j) for j in range(_LIMIT)]

    rows = [n] + [None] * (_LIMIT - 1)
    swin = n
    off = 0
    worst_ref[:, :s_len] = n / (jnp.maximum(jnp.sqrt(n), _EPS) * gns[0])
    incl_ref[:, :s_len] = jnp.ones_like(rmf)
    off += s_len

    regw = rmf
    for L in range(2, _LIMIT + 1):
        for j in range(L - 1):
            rows[j] = rows[j] + _shift(a[L - 1 - j], j)
        new = a[L - 1]
        for k in range(1, L):
            new = new + _shift(a[L - 1 - k], k)
        swin = swin + 2.0 * new - _shift(n, L - 1)
        rows[L - 1] = new
        regw = regw * _shift(rmf, L - 1)

        cn_inv = 1.0 / jnp.maximum(jnp.sqrt(jnp.maximum(swin, 0.0)),
                                   L * _EPS)
        worst = None
        for j in range(L):
            s_j = rows[j] * cn_inv / gns[j]
            worst = s_j if worst is None else jnp.minimum(worst, s_j)
        w = s_len - L + 1
        worst_ref[:, off:off + w] = worst[:, :w]
        incl_ref[:, off:off + w] = jnp.where(
            (worst >= _THRESHOLD) & (regw == 1), 1, 0)[:, :w]
        off += w


def kernel(batch_sequence_tensors, regular_tokens_mask):
    x = batch_sequence_tensors
    rm = regular_tokens_mask.astype(jnp.int32)
    b, s_len, _ = x.shape
    n_out = _LIMIT * s_len - (_LIMIT * (_LIMIT - 1)) // 2

    worst_all, incl = pl.pallas_call(
        _fused_kernel,
        grid=(b,),
        in_specs=[
            pl.BlockSpec((1, s_len, x.shape[2]), lambda i: (i, 0, 0)),
            pl.BlockSpec((b, s_len), lambda i: (0, 0)),
        ],
        out_specs=(
            pl.BlockSpec((b, n_out), lambda i: (0, 0)),
            pl.BlockSpec((b, n_out), lambda i: (0, 0)),
        ),
        out_shape=(
            jax.ShapeDtypeStruct((b, n_out), jnp.float32),
            jax.ShapeDtypeStruct((b, n_out), jnp.int32),
        ),
        scratch_shapes=[pltpu.VMEM((b, s_len, _LIMIT), jnp.float32)],
    )(x, rm)

    return worst_all, incl != 0
